# baseline (speedup 1.0000x reference)
import jax
import jax.numpy as jnp
from jax.experimental import pallas as pl
from jax.experimental.pallas import tpu as pltpu

NB = 25

def _dma_kernel(r_ref, o_ref):
    o_ref[...] = r_ref[0, :32, :]

def kernel(adj, recovery_stage_idx, preferred_type_idx, resource_type_idx,
           user_emb_w, item_emb_w, recovery_emb_w, type_emb_w,
           resource_type_emb_w, user_proj_w, user_proj_b,
           item_proj_w, item_proj_b):
    U, I = adj.shape
    br = U * I // 128 // NB
    r = adj.reshape(NB, br, 128)
    u_out = pl.pallas_call(
        _dma_kernel,
        grid=(NB,),
        in_specs=[pl.BlockSpec((1, br, 128), lambda i: (i, 0, 0))],
        out_specs=pl.BlockSpec((32, 128), lambda i: (0, 0)),
        out_shape=jax.ShapeDtypeStruct((32, 128), jnp.float32),
        compiler_params=pltpu.CompilerParams(
            dimension_semantics=("arbitrary",),
        ),
    )(r)
    return (jnp.zeros((U, 32), jnp.float32) + u_out[0, 0], jnp.zeros((I, 32), jnp.float32))


# speedup vs baseline: 3.1294x; 3.1294x over previous
import jax
import jax.numpy as jnp
from jax import lax
from jax.experimental import pallas as pl
from jax.experimental.pallas import tpu as pltpu

BU = 400


def _enrich_kernel(rec_idx_ref, typ_idx_ref, res_idx_ref,
                   user_emb_ref, item_emb_ref,
                   rec_w_ref, typ_w_ref, res_w_ref,
                   upw_ref, upb_ref, ipw_ref, ipb_ref,
                   eu_ref, ei_ref):
    U, D = user_emb_ref.shape
    I = item_emb_ref.shape[0]
    REC, F = rec_w_ref.shape
    TYP = typ_w_ref.shape[0]
    RES = res_w_ref.shape[0]

    def onehot(idx_col, n, rows):
        return (idx_col == lax.broadcasted_iota(jnp.int32, (rows, n), 1)
                ).astype(jnp.float32)

    rec_e = jnp.dot(onehot(rec_idx_ref[...], REC, U), rec_w_ref[...],
                    preferred_element_type=jnp.float32)
    typ_e = jnp.dot(onehot(typ_idx_ref[...], TYP, U), typ_w_ref[...],
                    preferred_element_type=jnp.float32)
    res_e = jnp.dot(onehot(res_idx_ref[...], RES, I), res_w_ref[...],
                    preferred_element_type=jnp.float32)

    def matmul_t(x, w):
        return lax.dot_general(x, w, (((1,), (1,)), ((), ())),
                               preferred_element_type=jnp.float32)

    upw = upw_ref[...]
    eu = (matmul_t(user_emb_ref[...], upw[:, :D])
          + matmul_t(rec_e, upw[:, D:D + F])
          + matmul_t(typ_e, upw[:, D + F:])
          + upb_ref[...])
    ipw = ipw_ref[...]
    ei = (matmul_t(item_emb_ref[...], ipw[:, :D])
          + matmul_t(res_e, ipw[:, D:])
          + ipb_ref[...])
    eu_ref[...] = eu
    ei_ref[...] = ei


def _layer1_kernel(eu_ref, ei_ref, adj_ref,
                   a16_ref, u1_ref, itT1_ref):
    i = pl.program_id(0)
    bu = adj_ref.shape[0]
    a = adj_ref[...].astype(jnp.bfloat16)
    a16_ref[...] = a
    row0 = i * bu
    u1_ref[pl.ds(row0, bu), :] = jnp.dot(
        a, ei_ref[...].astype(jnp.bfloat16),
        preferred_element_type=jnp.float32)
    contribT = lax.dot_general(
        eu_ref[pl.ds(row0, bu), :].astype(jnp.bfloat16), a,
        (((0,), (0,)), ((), ())),
        preferred_element_type=jnp.float32)

    @pl.when(i == 0)
    def _first():
        itT1_ref[...] = contribT

    @pl.when(i > 0)
    def _acc():
        itT1_ref[...] = itT1_ref[...] + contribT


def _prop23_kernel(eu_ref, ei_ref, u1_ref, itT1_ref, a16_ref,
                   u_out_ref, it_out_ref,
                   u_cur, it_cur, u_nxt, itT_nxt):
    l = pl.program_id(0)
    i = pl.program_id(1)
    nl = pl.num_programs(0)
    ni = pl.num_programs(1)
    bu = a16_ref.shape[0]

    @pl.when(jnp.logical_and(l == 0, i == 0))
    def _init():
        it1 = jnp.transpose(itT1_ref[...])
        u_cur[...] = u1_ref[...]
        it_cur[...] = it1
        u_out_ref[...] = eu_ref[...] + u1_ref[...]
        it_out_ref[...] = ei_ref[...] + it1

    a = a16_ref[...]
    row0 = i * bu
    u_nxt[pl.ds(row0, bu), :] = jnp.dot(
        a, it_cur[...].astype(jnp.bfloat16),
        preferred_element_type=jnp.float32)
    contribT = lax.dot_general(
        u_cur[pl.ds(row0, bu), :].astype(jnp.bfloat16), a,
        (((0,), (0,)), ((), ())),
        preferred_element_type=jnp.float32)

    @pl.when(i == 0)
    def _first():
        itT_nxt[...] = contribT

    @pl.when(i > 0)
    def _acc():
        itT_nxt[...] = itT_nxt[...] + contribT

    @pl.when(i == ni - 1)
    def _layer_end():
        it_new = jnp.transpose(itT_nxt[...])
        u_out_ref[...] = u_out_ref[...] + u_nxt[...]
        it_out_ref[...] = it_out_ref[...] + it_new
        u_cur[...] = u_nxt[...]
        it_cur[...] = it_new

    @pl.when(jnp.logical_and(l == nl - 1, i == ni - 1))
    def _finish():
        u_out_ref[...] = u_out_ref[...] * 0.25
        it_out_ref[...] = it_out_ref[...] * 0.25


def kernel(adj, recovery_stage_idx, preferred_type_idx, resource_type_idx,
           user_emb_w, item_emb_w, recovery_emb_w, type_emb_w,
           resource_type_emb_w, user_proj_w, user_proj_b,
           item_proj_w, item_proj_b):
    U, I = adj.shape
    D = user_emb_w.shape[1]
    f32 = jnp.float32

    rec_idx = recovery_stage_idx.astype(jnp.int32).reshape(U, 1)
    typ_idx = preferred_type_idx.astype(jnp.int32).reshape(U, 1)
    res_idx = resource_type_idx.astype(jnp.int32).reshape(I, 1)

    eu, ei = pl.pallas_call(
        _enrich_kernel,
        out_shape=[jax.ShapeDtypeStruct((U, D), f32),
                   jax.ShapeDtypeStruct((I, D), f32)],
    )(rec_idx, typ_idx, res_idx, user_emb_w, item_emb_w,
      recovery_emb_w, type_emb_w, resource_type_emb_w,
      user_proj_w, user_proj_b.reshape(1, D),
      item_proj_w, item_proj_b.reshape(1, D))

    ni = U // BU
    a16, u1, itT1 = pl.pallas_call(
        _layer1_kernel,
        grid=(ni,),
        in_specs=[
            pl.BlockSpec((U, D), lambda i: (0, 0)),
            pl.BlockSpec((I, D), lambda i: (0, 0)),
            pl.BlockSpec((BU, I), lambda i: (i, 0)),
        ],
        out_specs=[
            pl.BlockSpec((BU, I), lambda i: (i, 0)),
            pl.BlockSpec((U, D), lambda i: (0, 0)),
            pl.BlockSpec((D, I), lambda i: (0, 0)),
        ],
        out_shape=[jax.ShapeDtypeStruct((U, I), jnp.bfloat16),
                   jax.ShapeDtypeStruct((U, D), f32),
                   jax.ShapeDtypeStruct((D, I), f32)],
        compiler_params=pltpu.CompilerParams(
            dimension_semantics=("arbitrary",),
        ),
    )(eu, ei, adj)

    u_out, it_out = pl.pallas_call(
        _prop23_kernel,
        grid=(2, ni),
        in_specs=[
            pl.BlockSpec((U, D), lambda l, i: (0, 0)),
            pl.BlockSpec((I, D), lambda l, i: (0, 0)),
            pl.BlockSpec((U, D), lambda l, i: (0, 0)),
            pl.BlockSpec((D, I), lambda l, i: (0, 0)),
            pl.BlockSpec((BU, I), lambda l, i: (i, 0)),
        ],
        out_specs=[
            pl.BlockSpec((U, D), lambda l, i: (0, 0)),
            pl.BlockSpec((I, D), lambda l, i: (0, 0)),
        ],
        out_shape=[jax.ShapeDtypeStruct((U, D), f32),
                   jax.ShapeDtypeStruct((I, D), f32)],
        scratch_shapes=[
            pltpu.VMEM((U, D), f32),
            pltpu.VMEM((I, D), f32),
            pltpu.VMEM((U, D), f32),
            pltpu.VMEM((D, I), f32),
        ],
        compiler_params=pltpu.CompilerParams(
            dimension_semantics=("arbitrary", "arbitrary"),
        ),
    )(eu, ei, u1, itT1, a16)

    return (u_out, it_out)


# speedup vs baseline: 3.3834x; 1.0812x over previous
import jax
import jax.numpy as jnp
from jax import lax
from jax.experimental import pallas as pl
from jax.experimental.pallas import tpu as pltpu

NUM_LAYERS = 3
BU = 1000


def _enrich_kernel(rec_idx_ref, typ_idx_ref, res_idx_ref,
                   user_emb_ref, item_emb_ref,
                   rec_w_ref, typ_w_ref, res_w_ref,
                   upw_ref, upb_ref, ipw_ref, ipb_ref,
                   eu_ref, ei_ref):
    U, D = user_emb_ref.shape
    I = item_emb_ref.shape[0]
    REC, F = rec_w_ref.shape
    TYP = typ_w_ref.shape[0]
    RES = res_w_ref.shape[0]

    def onehot(idx_col, n, rows):
        return (idx_col == lax.broadcasted_iota(jnp.int32, (rows, n), 1)
                ).astype(jnp.float32)

    rec_e = jnp.dot(onehot(rec_idx_ref[...], REC, U), rec_w_ref[...],
                    preferred_element_type=jnp.float32)
    typ_e = jnp.dot(onehot(typ_idx_ref[...], TYP, U), typ_w_ref[...],
                    preferred_element_type=jnp.float32)
    res_e = jnp.dot(onehot(res_idx_ref[...], RES, I), res_w_ref[...],
                    preferred_element_type=jnp.float32)

    def matmul_t(x, w):
        return lax.dot_general(x, w, (((1,), (1,)), ((), ())),
                               preferred_element_type=jnp.float32)

    upw = upw_ref[...]
    eu = (matmul_t(user_emb_ref[...], upw[:, :D])
          + matmul_t(rec_e, upw[:, D:D + F])
          + matmul_t(typ_e, upw[:, D + F:])
          + upb_ref[...])
    ipw = ipw_ref[...]
    ei = (matmul_t(item_emb_ref[...], ipw[:, :D])
          + matmul_t(res_e, ipw[:, D:])
          + ipb_ref[...])
    eu_ref[...] = eu
    ei_ref[...] = ei


def _prop_kernel(eu_ref, ei_ref, adj_ref, u_out_ref, it_out_ref,
                 u_cur, it_cur, u_nxt, itT_nxt):
    l = pl.program_id(0)
    i = pl.program_id(1)
    nl = pl.num_programs(0)
    ni = pl.num_programs(1)
    bu = adj_ref.shape[0]

    @pl.when(jnp.logical_and(l == 0, i == 0))
    def _init():
        u_cur[...] = eu_ref[...]
        it_cur[...] = ei_ref[...]
        u_out_ref[...] = eu_ref[...]
        it_out_ref[...] = ei_ref[...]

    a = adj_ref[...]
    row0 = i * bu
    u_nxt[pl.ds(row0, bu), :] = jnp.dot(
        a, it_cur[...].astype(jnp.bfloat16),
        preferred_element_type=jnp.float32)
    contribT = lax.dot_general(
        u_cur[pl.ds(row0, bu), :].astype(jnp.bfloat16), a,
        (((0,), (0,)), ((), ())),
        preferred_element_type=jnp.float32)

    @pl.when(i == 0)
    def _first():
        itT_nxt[...] = contribT

    @pl.when(i > 0)
    def _acc():
        itT_nxt[...] = itT_nxt[...] + contribT

    @pl.when(i == ni - 1)
    def _layer_end():
        it_new = jnp.transpose(itT_nxt[...])
        u_out_ref[...] = u_out_ref[...] + u_nxt[...]
        it_out_ref[...] = it_out_ref[...] + it_new
        u_cur[...] = u_nxt[...]
        it_cur[...] = it_new

    @pl.when(jnp.logical_and(l == nl - 1, i == ni - 1))
    def _finish():
        scale = 1.0 / (nl + 1)
        u_out_ref[...] = u_out_ref[...] * scale
        it_out_ref[...] = it_out_ref[...] * scale


def kernel(adj, recovery_stage_idx, preferred_type_idx, resource_type_idx,
           user_emb_w, item_emb_w, recovery_emb_w, type_emb_w,
           resource_type_emb_w, user_proj_w, user_proj_b,
           item_proj_w, item_proj_b):
    U, I = adj.shape
    D = user_emb_w.shape[1]
    f32 = jnp.float32

    rec_idx = recovery_stage_idx.astype(jnp.int32).reshape(U, 1)
    typ_idx = preferred_type_idx.astype(jnp.int32).reshape(U, 1)
    res_idx = resource_type_idx.astype(jnp.int32).reshape(I, 1)

    eu, ei = pl.pallas_call(
        _enrich_kernel,
        out_shape=[jax.ShapeDtypeStruct((U, D), f32),
                   jax.ShapeDtypeStruct((I, D), f32)],
    )(rec_idx, typ_idx, res_idx, user_emb_w, item_emb_w,
      recovery_emb_w, type_emb_w, resource_type_emb_w,
      user_proj_w, user_proj_b.reshape(1, D),
      item_proj_w, item_proj_b.reshape(1, D))

    adj16 = adj.astype(jnp.bfloat16)

    ni = U // BU
    u_out, it_out = pl.pallas_call(
        _prop_kernel,
        grid=(NUM_LAYERS, ni),
        in_specs=[
            pl.BlockSpec((U, D), lambda l, i: (0, 0)),
            pl.BlockSpec((I, D), lambda l, i: (0, 0)),
            pl.BlockSpec((BU, I), lambda l, i: (i, 0)),
        ],
        out_specs=[
            pl.BlockSpec((U, D), lambda l, i: (0, 0)),
            pl.BlockSpec((I, D), lambda l, i: (0, 0)),
        ],
        out_shape=[jax.ShapeDtypeStruct((U, D), f32),
                   jax.ShapeDtypeStruct((I, D), f32)],
        scratch_shapes=[
            pltpu.VMEM((U, D), f32),
            pltpu.VMEM((I, D), f32),
            pltpu.VMEM((U, D), f32),
            pltpu.VMEM((D, I), f32),
        ],
        compiler_params=pltpu.CompilerParams(
            dimension_semantics=("arbitrary", "arbitrary"),
        ),
    )(eu, ei, adj16)

    return (u_out, it_out)
